# Initial kernel scaffold; baseline (speedup 1.0000x reference)
import jax
import jax.numpy as jnp
from jax.experimental import pallas as pl


def kernel(x, edge_index, W1, b1, g1, bt1, L1, lb1, W2, b2, g2, bt2, L2, lb2, W3, b3, g3, bt3, L3, lb3):
    raise NotImplementedError("write your pallas kernel here")


# speedup vs baseline: 19.6710x; 19.6710x over previous
import functools

import jax
import jax.numpy as jnp
from jax import lax
from jax.experimental import pallas as pl
from jax.experimental.pallas import tpu as pltpu
from jax.experimental.pallas import tpu_sc as plsc

N = 10000
E = 320000
HID = 64
NCORE = 2
NSUB = 16
NW = NCORE * NSUB
NPAD = 10240
SLICE = NPAD // NSUB
C = 80
EPW = E // NW
NCH = EPW // C
DW = 16
RB = 1000
GRID = N // RB

_mesh = plsc.VectorSubcoreMesh(core_axis_name="c", subcore_axis_name="s")
_sc_params = pltpu.CompilerParams(use_tc_tiling_on_sc=False)


def _sc_degree(dst2, ones_u, zrows):
    @functools.partial(
        pl.kernel,
        out_type=jax.ShapeDtypeStruct((NCORE, NPAD, DW), jnp.float32),
        mesh=_mesh,
        compiler_params=_sc_params,
        scratch_types=[
            pltpu.VMEM((NCH, C), jnp.int32),
            pltpu.VMEM((C, DW), jnp.float32),
            pltpu.VMEM_SHARED((NPAD, DW), jnp.float32),
        ],
    )
    def k(dst_hbm, ones_hbm, z_hbm, out_hbm, didx, ones_v, acc):
        c = lax.axis_index("c")
        s = lax.axis_index("s")
        w = c * NSUB + s
        pltpu.sync_copy(z_hbm, acc.at[pl.ds(s * SLICE, SLICE)])
        pltpu.sync_copy(dst_hbm.at[w], didx)
        pltpu.sync_copy(ones_hbm, ones_v)
        plsc.subcore_barrier()

        @pl.loop(0, NCH)
        def _(j):
            pltpu.sync_copy(ones_v, acc.at[didx.at[j]], add=True)

        plsc.subcore_barrier()
        pltpu.sync_copy(acc.at[pl.ds(s * SLICE, SLICE)],
                        out_hbm.at[c, pl.ds(s * SLICE, SLICE)])

    return k(dst2, ones_u, zrows)


def _sc_scatter(hs, src2, dst2, zrows):
    @functools.partial(
        pl.kernel,
        out_type=jax.ShapeDtypeStruct((NCORE, NPAD, HID), jnp.float32),
        mesh=_mesh,
        compiler_params=_sc_params,
        scratch_types=[
            pltpu.VMEM((NCH, C), jnp.int32),
            pltpu.VMEM((NCH, C), jnp.int32),
            pltpu.VMEM((C, HID), jnp.float32),
            pltpu.VMEM_SHARED((NPAD, HID), jnp.float32),
        ],
    )
    def k(hs_hbm, src_hbm, dst_hbm, z_hbm, out_hbm, sidx, didx, rows, acc):
        c = lax.axis_index("c")
        s = lax.axis_index("s")
        w = c * NSUB + s
        pltpu.sync_copy(z_hbm, acc.at[pl.ds(s * SLICE, SLICE)])
        pltpu.sync_copy(src_hbm.at[w], sidx)
        pltpu.sync_copy(dst_hbm.at[w], didx)
        plsc.subcore_barrier()

        @pl.loop(0, NCH)
        def _(j):
            pltpu.sync_copy(hs_hbm.at[sidx.at[j]], rows)
            pltpu.sync_copy(rows, acc.at[didx.at[j]], add=True)

        plsc.subcore_barrier()
        pltpu.sync_copy(acc.at[pl.ds(s * SLICE, SLICE)],
                        out_hbm.at[c, pl.ds(s * SLICE, SLICE)])

    return k(hs, src2, dst2, zrows)


def _tc_mm(x, W):
    n, d = x.shape
    _, h = W.shape

    def body(x_ref, w_ref, o_ref):
        o_ref[...] = jnp.dot(x_ref[...], w_ref[...],
                             preferred_element_type=jnp.float32)

    return pl.pallas_call(
        body,
        grid=(GRID,),
        in_specs=[
            pl.BlockSpec((RB, d), lambda i: (i, 0)),
            pl.BlockSpec((d, h), lambda i: (0, 0)),
        ],
        out_specs=pl.BlockSpec((RB, h), lambda i: (i, 0)),
        out_shape=jax.ShapeDtypeStruct((n, h), jnp.float32),
    )(x, W)


def _tc_prescale(degp, h1):
    def body(d_ref, h_ref, dinv_ref, hs_ref):
        deg = d_ref[0] + d_ref[1] + 1.0
        dv = lax.rsqrt(deg)
        dinv_ref[...] = dv
        hs_ref[...] = h_ref[...] * dv[:, 0:1]

    return pl.pallas_call(
        body,
        grid=(GRID,),
        in_specs=[
            pl.BlockSpec((NCORE, RB, DW), lambda i: (0, i, 0)),
            pl.BlockSpec((RB, HID), lambda i: (i, 0)),
        ],
        out_specs=[
            pl.BlockSpec((RB, DW), lambda i: (i, 0)),
            pl.BlockSpec((RB, HID), lambda i: (i, 0)),
        ],
        out_shape=[
            jax.ShapeDtypeStruct((N, DW), jnp.float32),
            jax.ShapeDtypeStruct((N, HID), jnp.float32),
        ],
    )(degp, h1)


def _post_math(S_ref, hs_ref, dinv_ref, b_ref, g_ref, bt_ref, L_ref, lb_ref):
    dv = dinv_ref[...][:, 0:1]
    agg = (S_ref[0] + S_ref[1] + hs_ref[...]) * dv + b_ref[...]
    h = jnp.maximum(agg, 0.0)
    mu = jnp.mean(h, axis=-1, keepdims=True)
    xc = h - mu
    var = jnp.mean(xc * xc, axis=-1, keepdims=True)
    hn = xc * lax.rsqrt(var + 1e-5) * g_ref[...] + bt_ref[...]
    h2 = jnp.dot(hn, L_ref[...], preferred_element_type=jnp.float32)
    return jnp.maximum(h2 + lb_ref[...], 0.0), dv


def _tc_post(S, hs, dinv, b, g, bt, L, lb, Wn):
    d2 = L.shape[1]

    def body(S_ref, hs_ref, dinv_ref, b_ref, g_ref, bt_ref, L_ref, lb_ref,
             wn_ref, o_ref):
        h2, dv = _post_math(S_ref, hs_ref, dinv_ref, b_ref, g_ref, bt_ref,
                            L_ref, lb_ref)
        o_ref[...] = jnp.dot(h2, wn_ref[...],
                             preferred_element_type=jnp.float32) * dv

    return pl.pallas_call(
        body,
        grid=(GRID,),
        in_specs=[
            pl.BlockSpec((NCORE, RB, HID), lambda i: (0, i, 0)),
            pl.BlockSpec((RB, HID), lambda i: (i, 0)),
            pl.BlockSpec((RB, DW), lambda i: (i, 0)),
            pl.BlockSpec((1, HID), lambda i: (0, 0)),
            pl.BlockSpec((1, HID), lambda i: (0, 0)),
            pl.BlockSpec((1, HID), lambda i: (0, 0)),
            pl.BlockSpec((HID, d2), lambda i: (0, 0)),
            pl.BlockSpec((1, d2), lambda i: (0, 0)),
            pl.BlockSpec((d2, HID), lambda i: (0, 0)),
        ],
        out_specs=pl.BlockSpec((RB, HID), lambda i: (i, 0)),
        out_shape=jax.ShapeDtypeStruct((N, HID), jnp.float32),
    )(S, hs, dinv, b, g, bt, L, lb, Wn)


def _tc_final(S, hs, dinv, b, g, bt, L, lb):
    d2 = L.shape[1]

    def body(S_ref, hs_ref, dinv_ref, b_ref, g_ref, bt_ref, L_ref, lb_ref,
             o_ref):
        h2, _ = _post_math(S_ref, hs_ref, dinv_ref, b_ref, g_ref, bt_ref,
                           L_ref, lb_ref)
        m = jnp.max(h2, axis=0, keepdims=True)
        i = pl.program_id(0)

        @pl.when(i == 0)
        def _():
            o_ref[...] = m

        @pl.when(i > 0)
        def _():
            o_ref[...] = jnp.maximum(o_ref[...], m)

    return pl.pallas_call(
        body,
        grid=(GRID,),
        in_specs=[
            pl.BlockSpec((NCORE, RB, HID), lambda i: (0, i, 0)),
            pl.BlockSpec((RB, HID), lambda i: (i, 0)),
            pl.BlockSpec((RB, DW), lambda i: (i, 0)),
            pl.BlockSpec((1, HID), lambda i: (0, 0)),
            pl.BlockSpec((1, HID), lambda i: (0, 0)),
            pl.BlockSpec((1, HID), lambda i: (0, 0)),
            pl.BlockSpec((HID, d2), lambda i: (0, 0)),
            pl.BlockSpec((1, d2), lambda i: (0, 0)),
        ],
        out_specs=pl.BlockSpec((1, d2), lambda i: (0, 0)),
        out_shape=jax.ShapeDtypeStruct((1, d2), jnp.float32),
    )(S, hs, dinv, b, g, bt, L, lb)


def kernel(x, edge_index,
           W1, b1, g1, bt1, L1, lb1,
           W2, b2, g2, bt2, L2, lb2,
           W3, b3, g3, bt3, L3, lb3):
    src2 = edge_index[0].reshape(NW, NCH, C)
    dst2 = edge_index[1].reshape(NW, NCH, C)

    z_deg = jnp.zeros((SLICE, DW), jnp.float32)
    z_acc = jnp.zeros((SLICE, HID), jnp.float32)
    ones_u = jnp.ones((C, DW), jnp.float32)

    row = lambda v: v.reshape(1, -1)

    degp = _sc_degree(dst2, ones_u, z_deg)
    h1 = _tc_mm(x, W1)
    dinv, hs = _tc_prescale(degp, h1)

    S = _sc_scatter(hs, src2, dst2, z_acc)
    hs = _tc_post(S, hs, dinv, row(b1), row(g1), row(bt1), L1, row(lb1), W2)

    S = _sc_scatter(hs, src2, dst2, z_acc)
    hs = _tc_post(S, hs, dinv, row(b2), row(g2), row(bt2), L2, row(lb2), W3)

    S = _sc_scatter(hs, src2, dst2, z_acc)
    out = _tc_final(S, hs, dinv, row(b3), row(g3), row(bt3), L3, row(lb3))
    return out.reshape(L3.shape[1])


# speedup vs baseline: 28.5514x; 1.4514x over previous
import functools

import jax
import jax.numpy as jnp
from jax import lax
from jax.experimental import pallas as pl
from jax.experimental.pallas import tpu as pltpu
from jax.experimental.pallas import tpu_sc as plsc

N = 10000
E = 320000
HID = 64
NCORE = 2
NSUB = 16
NW = NCORE * NSUB
NPAD = 10240
SLICE = NPAD // NSUB
C = 80
EPW = E // NW
NCH = EPW // C
DW = 16
RB = 1000
GRID = N // RB

_mesh = plsc.VectorSubcoreMesh(core_axis_name="c", subcore_axis_name="s")
_sc_params = pltpu.CompilerParams(use_tc_tiling_on_sc=False)


def _sc_degree(dst2, ones_u, zrows):
    @functools.partial(
        pl.kernel,
        out_type=jax.ShapeDtypeStruct((NCORE, NPAD, DW), jnp.float32),
        mesh=_mesh,
        compiler_params=_sc_params,
        scratch_types=[
            pltpu.VMEM((NCH, C), jnp.int32),
            pltpu.VMEM((C, DW), jnp.float32),
            pltpu.VMEM_SHARED((NPAD, DW), jnp.float32),
        ],
    )
    def k(dst_hbm, ones_hbm, z_hbm, out_hbm, didx, ones_v, acc):
        c = lax.axis_index("c")
        s = lax.axis_index("s")
        w = c * NSUB + s
        pltpu.sync_copy(z_hbm, acc.at[pl.ds(s * SLICE, SLICE)])
        pltpu.sync_copy(dst_hbm.at[w], didx)
        pltpu.sync_copy(ones_hbm, ones_v)
        plsc.subcore_barrier()

        @pl.loop(0, NCH)
        def _(j):
            pltpu.sync_copy(ones_v, acc.at[didx.at[j]], add=True)

        plsc.subcore_barrier()
        pltpu.sync_copy(acc.at[pl.ds(s * SLICE, SLICE)],
                        out_hbm.at[c, pl.ds(s * SLICE, SLICE)])

    return k(dst2, ones_u, zrows)


def _sc_scatter(hs, src2, dst2, zrows):
    @functools.partial(
        pl.kernel,
        out_type=jax.ShapeDtypeStruct((NCORE, NPAD, HID), jnp.float32),
        mesh=_mesh,
        compiler_params=_sc_params,
        scratch_types=[
            pltpu.VMEM((NCH, C), jnp.int32),
            pltpu.VMEM((NCH, C), jnp.int32),
            pltpu.VMEM((C, HID), jnp.float32),
            pltpu.VMEM((C, HID), jnp.float32),
            pltpu.SemaphoreType.DMA,
            pltpu.SemaphoreType.DMA,
            pltpu.VMEM_SHARED((NPAD, HID), jnp.float32),
        ],
    )
    def k(hs_hbm, src_hbm, dst_hbm, z_hbm, out_hbm, sidx, didx,
          rows_a, rows_b, sem_a, sem_b, acc):
        c = lax.axis_index("c")
        s = lax.axis_index("s")
        w = c * NSUB + s
        pltpu.sync_copy(z_hbm, acc.at[pl.ds(s * SLICE, SLICE)])
        pltpu.sync_copy(src_hbm.at[w], sidx)
        pltpu.sync_copy(dst_hbm.at[w], didx)
        plsc.subcore_barrier()

        def fire(j, rows, sem):
            pltpu.async_copy(hs_hbm.at[sidx.at[j]], rows, sem)

        def drain(j, rows, sem):
            pltpu.make_async_copy(hs_hbm.at[sidx.at[j]], rows, sem).wait()

        fire(0, rows_a, sem_a)

        @pl.loop(0, NCH - 1, step=2)
        def _(j):
            fire(j + 1, rows_b, sem_b)
            drain(j, rows_a, sem_a)
            pltpu.sync_copy(rows_a, acc.at[didx.at[j]], add=True)
            fire(j + 2, rows_a, sem_a)
            drain(j + 1, rows_b, sem_b)
            pltpu.sync_copy(rows_b, acc.at[didx.at[j + 1]], add=True)

        drain(NCH - 1, rows_a, sem_a)
        pltpu.sync_copy(rows_a, acc.at[didx.at[NCH - 1]], add=True)
        plsc.subcore_barrier()
        pltpu.sync_copy(acc.at[pl.ds(s * SLICE, SLICE)],
                        out_hbm.at[c, pl.ds(s * SLICE, SLICE)])

    return k(hs, src2, dst2, zrows)


def _tc_mm(x, W):
    n, d = x.shape
    _, h = W.shape

    def body(x_ref, w_ref, o_ref):
        o_ref[...] = jnp.dot(x_ref[...], w_ref[...],
                             preferred_element_type=jnp.float32)

    return pl.pallas_call(
        body,
        grid=(GRID,),
        in_specs=[
            pl.BlockSpec((RB, d), lambda i: (i, 0)),
            pl.BlockSpec((d, h), lambda i: (0, 0)),
        ],
        out_specs=pl.BlockSpec((RB, h), lambda i: (i, 0)),
        out_shape=jax.ShapeDtypeStruct((n, h), jnp.float32),
    )(x, W)


def _tc_prescale(degp, h1):
    def body(d_ref, h_ref, dinv_ref, hs_ref):
        deg = d_ref[0] + d_ref[1] + 1.0
        dv = lax.rsqrt(deg)
        dinv_ref[...] = dv
        hs_ref[...] = h_ref[...] * dv[:, 0:1]

    return pl.pallas_call(
        body,
        grid=(GRID,),
        in_specs=[
            pl.BlockSpec((NCORE, RB, DW), lambda i: (0, i, 0)),
            pl.BlockSpec((RB, HID), lambda i: (i, 0)),
        ],
        out_specs=[
            pl.BlockSpec((RB, DW), lambda i: (i, 0)),
            pl.BlockSpec((RB, HID), lambda i: (i, 0)),
        ],
        out_shape=[
            jax.ShapeDtypeStruct((N, DW), jnp.float32),
            jax.ShapeDtypeStruct((N, HID), jnp.float32),
        ],
    )(degp, h1)


def _post_math(S_ref, hs_ref, dinv_ref, b_ref, g_ref, bt_ref, L_ref, lb_ref):
    dv = dinv_ref[...][:, 0:1]
    agg = (S_ref[0] + S_ref[1] + hs_ref[...]) * dv + b_ref[...]
    h = jnp.maximum(agg, 0.0)
    mu = jnp.mean(h, axis=-1, keepdims=True)
    xc = h - mu
    var = jnp.mean(xc * xc, axis=-1, keepdims=True)
    hn = xc * lax.rsqrt(var + 1e-5) * g_ref[...] + bt_ref[...]
    h2 = jnp.dot(hn, L_ref[...], preferred_element_type=jnp.float32)
    return jnp.maximum(h2 + lb_ref[...], 0.0), dv


def _tc_post(S, hs, dinv, b, g, bt, L, lb, Wn):
    d2 = L.shape[1]

    def body(S_ref, hs_ref, dinv_ref, b_ref, g_ref, bt_ref, L_ref, lb_ref,
             wn_ref, o_ref):
        h2, dv = _post_math(S_ref, hs_ref, dinv_ref, b_ref, g_ref, bt_ref,
                            L_ref, lb_ref)
        o_ref[...] = jnp.dot(h2, wn_ref[...],
                             preferred_element_type=jnp.float32) * dv

    return pl.pallas_call(
        body,
        grid=(GRID,),
        in_specs=[
            pl.BlockSpec((NCORE, RB, HID), lambda i: (0, i, 0)),
            pl.BlockSpec((RB, HID), lambda i: (i, 0)),
            pl.BlockSpec((RB, DW), lambda i: (i, 0)),
            pl.BlockSpec((1, HID), lambda i: (0, 0)),
            pl.BlockSpec((1, HID), lambda i: (0, 0)),
            pl.BlockSpec((1, HID), lambda i: (0, 0)),
            pl.BlockSpec((HID, d2), lambda i: (0, 0)),
            pl.BlockSpec((1, d2), lambda i: (0, 0)),
            pl.BlockSpec((d2, HID), lambda i: (0, 0)),
        ],
        out_specs=pl.BlockSpec((RB, HID), lambda i: (i, 0)),
        out_shape=jax.ShapeDtypeStruct((N, HID), jnp.float32),
    )(S, hs, dinv, b, g, bt, L, lb, Wn)


def _tc_final(S, hs, dinv, b, g, bt, L, lb):
    d2 = L.shape[1]

    def body(S_ref, hs_ref, dinv_ref, b_ref, g_ref, bt_ref, L_ref, lb_ref,
             o_ref):
        h2, _ = _post_math(S_ref, hs_ref, dinv_ref, b_ref, g_ref, bt_ref,
                           L_ref, lb_ref)
        m = jnp.max(h2, axis=0, keepdims=True)
        i = pl.program_id(0)

        @pl.when(i == 0)
        def _():
            o_ref[...] = m

        @pl.when(i > 0)
        def _():
            o_ref[...] = jnp.maximum(o_ref[...], m)

    return pl.pallas_call(
        body,
        grid=(GRID,),
        in_specs=[
            pl.BlockSpec((NCORE, RB, HID), lambda i: (0, i, 0)),
            pl.BlockSpec((RB, HID), lambda i: (i, 0)),
            pl.BlockSpec((RB, DW), lambda i: (i, 0)),
            pl.BlockSpec((1, HID), lambda i: (0, 0)),
            pl.BlockSpec((1, HID), lambda i: (0, 0)),
            pl.BlockSpec((1, HID), lambda i: (0, 0)),
            pl.BlockSpec((HID, d2), lambda i: (0, 0)),
            pl.BlockSpec((1, d2), lambda i: (0, 0)),
        ],
        out_specs=pl.BlockSpec((1, d2), lambda i: (0, 0)),
        out_shape=jax.ShapeDtypeStruct((1, d2), jnp.float32),
    )(S, hs, dinv, b, g, bt, L, lb)


def kernel(x, edge_index,
           W1, b1, g1, bt1, L1, lb1,
           W2, b2, g2, bt2, L2, lb2,
           W3, b3, g3, bt3, L3, lb3):
    src2 = edge_index[0].reshape(NW, NCH, C)
    dst2 = edge_index[1].reshape(NW, NCH, C)

    z_deg = jnp.zeros((SLICE, DW), jnp.float32)
    z_acc = jnp.zeros((SLICE, HID), jnp.float32)
    ones_u = jnp.ones((C, DW), jnp.float32)

    row = lambda v: v.reshape(1, -1)

    degp = _sc_degree(dst2, ones_u, z_deg)
    h1 = _tc_mm(x, W1)
    dinv, hs = _tc_prescale(degp, h1)

    S = _sc_scatter(hs, src2, dst2, z_acc)
    hs = _tc_post(S, hs, dinv, row(b1), row(g1), row(bt1), L1, row(lb1), W2)

    S = _sc_scatter(hs, src2, dst2, z_acc)
    hs = _tc_post(S, hs, dinv, row(b2), row(g2), row(bt2), L2, row(lb2), W3)

    S = _sc_scatter(hs, src2, dst2, z_acc)
    out = _tc_final(S, hs, dinv, row(b3), row(g3), row(bt3), L3, row(lb3))
    return out.reshape(L3.shape[1])


# speedup vs baseline: 32.2886x; 1.1309x over previous
import functools

import jax
import jax.numpy as jnp
from jax import lax
from jax.experimental import pallas as pl
from jax.experimental.pallas import tpu as pltpu
from jax.experimental.pallas import tpu_sc as plsc

N = 10000
E = 320000
HID = 64
NCORE = 2
NSUB = 16
NW = NCORE * NSUB
NPAD = 10240
SLICE = NPAD // NSUB
C = 125
EPW = E // NW
NCH = EPW // C
DW = 16
RB = 1000
GRID = N // RB

_mesh = plsc.VectorSubcoreMesh(core_axis_name="c", subcore_axis_name="s")
_sc_params = pltpu.CompilerParams(use_tc_tiling_on_sc=False)


def _sc_degree(edge3, ones_u, zrows):
    @functools.partial(
        pl.kernel,
        out_type=jax.ShapeDtypeStruct((NCORE, NPAD, DW), jnp.float32),
        mesh=_mesh,
        compiler_params=_sc_params,
        scratch_types=[
            pltpu.VMEM((NCH, C), jnp.int32),
            pltpu.VMEM((C, DW), jnp.float32),
            pltpu.VMEM_SHARED((NPAD, DW), jnp.float32),
        ],
    )
    def k(edge_hbm, ones_hbm, z_hbm, out_hbm, didx, ones_v, acc):
        c = lax.axis_index("c")
        s = lax.axis_index("s")
        w = c * NSUB + s
        pltpu.sync_copy(z_hbm, acc.at[pl.ds(s * SLICE, SLICE)])
        pltpu.sync_copy(edge_hbm.at[1, w], didx)
        pltpu.sync_copy(ones_hbm, ones_v)
        plsc.subcore_barrier()

        @pl.loop(0, NCH)
        def _(j):
            pltpu.sync_copy(ones_v, acc.at[didx.at[j]], add=True)

        plsc.subcore_barrier()
        pltpu.sync_copy(acc.at[pl.ds(s * SLICE, SLICE)],
                        out_hbm.at[c, pl.ds(s * SLICE, SLICE)])

    return k(edge3, ones_u, zrows)


def _sc_scatter(hs, edge3, zrows):
    @functools.partial(
        pl.kernel,
        out_type=jax.ShapeDtypeStruct((NCORE, NPAD, HID), jnp.float32),
        mesh=_mesh,
        scratch_types=[
            pltpu.VMEM((NCH, C), jnp.int32),
            pltpu.VMEM((NCH, C), jnp.int32),
            pltpu.VMEM((C, HID), jnp.float32),
            pltpu.VMEM((C, HID), jnp.float32),
            pltpu.SemaphoreType.DMA,
            pltpu.SemaphoreType.DMA,
            pltpu.VMEM_SHARED((NPAD, HID), jnp.float32),
        ],
        compiler_params=_sc_params,
    )
    def k(hs_hbm, edge_hbm, z_hbm, out_hbm, sidx, didx,
          rows_a, rows_b, sem_a, sem_b, acc):
        c = lax.axis_index("c")
        s = lax.axis_index("s")
        w = c * NSUB + s
        pltpu.sync_copy(z_hbm, acc.at[pl.ds(s * SLICE, SLICE)])
        pltpu.sync_copy(edge_hbm.at[0, w], sidx)
        pltpu.sync_copy(edge_hbm.at[1, w], didx)
        plsc.subcore_barrier()

        def fire(j, rows, sem):
            pltpu.async_copy(hs_hbm.at[sidx.at[j]], rows, sem)

        def drain(j, rows, sem):
            pltpu.make_async_copy(hs_hbm.at[sidx.at[j]], rows, sem).wait()

        fire(0, rows_a, sem_a)

        @pl.loop(0, NCH, step=2)
        def _(j):
            fire(j + 1, rows_b, sem_b)
            drain(j, rows_a, sem_a)
            pltpu.sync_copy(rows_a, acc.at[didx.at[j]], add=True)

            @pl.when(j + 2 < NCH)
            def _():
                fire(j + 2, rows_a, sem_a)

            drain(j + 1, rows_b, sem_b)
            pltpu.sync_copy(rows_b, acc.at[didx.at[j + 1]], add=True)

        plsc.subcore_barrier()
        pltpu.sync_copy(acc.at[pl.ds(s * SLICE, SLICE)],
                        out_hbm.at[c, pl.ds(s * SLICE, SLICE)])

    return k(hs, edge3, zrows)


def _tc_mm(x, W):
    n, d = x.shape
    _, h = W.shape

    def body(x_ref, w_ref, o_ref):
        o_ref[...] = jnp.dot(x_ref[...], w_ref[...],
                             preferred_element_type=jnp.float32)

    return pl.pallas_call(
        body,
        grid=(GRID,),
        in_specs=[
            pl.BlockSpec((RB, d), lambda i: (i, 0)),
            pl.BlockSpec((d, h), lambda i: (0, 0)),
        ],
        out_specs=pl.BlockSpec((RB, h), lambda i: (i, 0)),
        out_shape=jax.ShapeDtypeStruct((n, h), jnp.float32),
    )(x, W)


def _tc_prescale(degp, h1):
    def body(d_ref, h_ref, dinv_ref, hs_ref):
        deg = d_ref[0] + d_ref[1] + 1.0
        dv = lax.rsqrt(deg)
        dinv_ref[...] = dv
        hs_ref[...] = h_ref[...] * dv[:, 0:1]

    return pl.pallas_call(
        body,
        grid=(GRID,),
        in_specs=[
            pl.BlockSpec((NCORE, RB, DW), lambda i: (0, i, 0)),
            pl.BlockSpec((RB, HID), lambda i: (i, 0)),
        ],
        out_specs=[
            pl.BlockSpec((RB, DW), lambda i: (i, 0)),
            pl.BlockSpec((RB, HID), lambda i: (i, 0)),
        ],
        out_shape=[
            jax.ShapeDtypeStruct((N, DW), jnp.float32),
            jax.ShapeDtypeStruct((N, HID), jnp.float32),
        ],
    )(degp, h1)


def _post_math(S_ref, hs_ref, dinv_ref, b_ref, g_ref, bt_ref, L_ref, lb_ref):
    dv = dinv_ref[...][:, 0:1]
    agg = (S_ref[0] + S_ref[1] + hs_ref[...]) * dv + b_ref[...]
    h = jnp.maximum(agg, 0.0)
    mu = jnp.mean(h, axis=-1, keepdims=True)
    xc = h - mu
    var = jnp.mean(xc * xc, axis=-1, keepdims=True)
    hn = xc * lax.rsqrt(var + 1e-5) * g_ref[...] + bt_ref[...]
    h2 = jnp.dot(hn, L_ref[...], preferred_element_type=jnp.float32)
    return jnp.maximum(h2 + lb_ref[...], 0.0), dv


def _tc_post(S, hs, dinv, b, g, bt, L, lb, Wn):
    d2 = L.shape[1]

    def body(S_ref, hs_ref, dinv_ref, b_ref, g_ref, bt_ref, L_ref, lb_ref,
             wn_ref, o_ref):
        h2, dv = _post_math(S_ref, hs_ref, dinv_ref, b_ref, g_ref, bt_ref,
                            L_ref, lb_ref)
        o_ref[...] = jnp.dot(h2, wn_ref[...],
                             preferred_element_type=jnp.float32) * dv

    return pl.pallas_call(
        body,
        grid=(GRID,),
        in_specs=[
            pl.BlockSpec((NCORE, RB, HID), lambda i: (0, i, 0)),
            pl.BlockSpec((RB, HID), lambda i: (i, 0)),
            pl.BlockSpec((RB, DW), lambda i: (i, 0)),
            pl.BlockSpec((1, HID), lambda i: (0, 0)),
            pl.BlockSpec((1, HID), lambda i: (0, 0)),
            pl.BlockSpec((1, HID), lambda i: (0, 0)),
            pl.BlockSpec((HID, d2), lambda i: (0, 0)),
            pl.BlockSpec((1, d2), lambda i: (0, 0)),
            pl.BlockSpec((d2, HID), lambda i: (0, 0)),
        ],
        out_specs=pl.BlockSpec((RB, HID), lambda i: (i, 0)),
        out_shape=jax.ShapeDtypeStruct((N, HID), jnp.float32),
    )(S, hs, dinv, b, g, bt, L, lb, Wn)


def _tc_final(S, hs, dinv, b, g, bt, L, lb):
    d2 = L.shape[1]

    def body(S_ref, hs_ref, dinv_ref, b_ref, g_ref, bt_ref, L_ref, lb_ref,
             o_ref):
        h2, _ = _post_math(S_ref, hs_ref, dinv_ref, b_ref, g_ref, bt_ref,
                           L_ref, lb_ref)
        m = jnp.max(h2, axis=0, keepdims=True)
        i = pl.program_id(0)

        @pl.when(i == 0)
        def _():
            o_ref[...] = m

        @pl.when(i > 0)
        def _():
            o_ref[...] = jnp.maximum(o_ref[...], m)

    return pl.pallas_call(
        body,
        grid=(GRID,),
        in_specs=[
            pl.BlockSpec((NCORE, RB, HID), lambda i: (0, i, 0)),
            pl.BlockSpec((RB, HID), lambda i: (i, 0)),
            pl.BlockSpec((RB, DW), lambda i: (i, 0)),
            pl.BlockSpec((1, HID), lambda i: (0, 0)),
            pl.BlockSpec((1, HID), lambda i: (0, 0)),
            pl.BlockSpec((1, HID), lambda i: (0, 0)),
            pl.BlockSpec((HID, d2), lambda i: (0, 0)),
            pl.BlockSpec((1, d2), lambda i: (0, 0)),
        ],
        out_specs=pl.BlockSpec((1, d2), lambda i: (0, 0)),
        out_shape=jax.ShapeDtypeStruct((1, d2), jnp.float32),
    )(S, hs, dinv, b, g, bt, L, lb)


def kernel(x, edge_index,
           W1, b1, g1, bt1, L1, lb1,
           W2, b2, g2, bt2, L2, lb2,
           W3, b3, g3, bt3, L3, lb3):
    edge3 = edge_index.reshape(2, NW, NCH, C)

    z_deg = jnp.zeros((SLICE, DW), jnp.float32)
    z_acc = jnp.zeros((SLICE, HID), jnp.float32)
    ones_u = jnp.ones((C, DW), jnp.float32)

    row = lambda v: v.reshape(1, -1)

    degp = _sc_degree(edge3, ones_u, z_deg)
    h1 = _tc_mm(x, W1)
    dinv, hs = _tc_prescale(degp, h1)

    S = _sc_scatter(hs, edge3, z_acc)
    hs = _tc_post(S, hs, dinv, row(b1), row(g1), row(bt1), L1, row(lb1), W2)

    S = _sc_scatter(hs, edge3, z_acc)
    hs = _tc_post(S, hs, dinv, row(b2), row(g2), row(bt2), L2, row(lb2), W3)

    S = _sc_scatter(hs, edge3, z_acc)
    out = _tc_final(S, hs, dinv, row(b3), row(g3), row(bt3), L3, row(lb3))
    return out.reshape(L3.shape[1])
